# baseline (speedup 1.0000x reference)
import jax
import jax.numpy as jnp
from jax.experimental import pallas as pl
from jax.experimental.pallas import tpu as pltpu

_CD = jnp.bfloat16
_NT = (((1,), (1,)), ((), ()))


def _make_body(R, L, B, Din, D0, D1):
    rel_lo = D0 + D1
    bias_lo = rel_lo + B

    def body(x_ref, *refs):
        adj_refs = refs[:R]
        pk_ref = refs[R]
        out_ref = refs[R + 1]
        wx0_s, wx1_s, relp0_s, relp1_s = refs[R + 2:]
        b = pl.program_id(0)

        @pl.when(b == 0)
        def _prep():
            for w_lo, din, wx_s, relp_s, D in (
                    (0, Din, wx0_s, relp0_s, D0),
                    (D0, D0, wx1_s, relp1_s, D1)):
                K = din + L
                for r in range(R):
                    wx_r = pk_ref[w_lo:w_lo + D, r * K:r * K + din]
                    wx_s[:, r * D:(r + 1) * D] = wx_r.astype(_CD).T
                    wrel_r = pk_ref[w_lo:w_lo + D,
                                    r * K + din:(r + 1) * K].astype(_CD)
                    rel_r = pk_ref[rel_lo:rel_lo + B,
                                   r * L:(r + 1) * L].astype(_CD)
                    relp_s[:, r * D:(r + 1) * D] = jax.lax.dot_general(
                        rel_r, wrel_r, _NT,
                        preferred_element_type=jnp.float32)

        adj_c = [a_ref[0, 0].astype(_CD) for a_ref in adj_refs]

        h = x_ref[0]
        for wx_s, relp_s, b_lo, D in ((wx0_s, relp0_s, 0, D0),
                                      (wx1_s, relp1_s, D0, D1)):
            y = jnp.dot(h.astype(_CD), wx_s[...],
                        preferred_element_type=jnp.float32)
            y = (y + relp_s[pl.ds(b, 1), :]).astype(_CD)
            acc = jnp.dot(adj_c[0], y[:, :D], preferred_element_type=jnp.float32)
            for r in range(1, R):
                acc += jnp.dot(adj_c[r], y[:, r * D:(r + 1) * D],
                               preferred_element_type=jnp.float32)
            bias = pk_ref[bias_lo:bias_lo + 1, b_lo:b_lo + D]
            h = jnp.maximum(acc + bias, 0.0)
        out_ref[0] = h
    return body


def kernel(node_features, relation_features, adj, w0, b0, w1, b1):
    B, N, Din = node_features.shape
    _, R, L = relation_features.shape
    D0, D1 = w0.shape[0], w1.shape[0]
    W = max(w0.shape[1], w1.shape[1])

    def pad_w(a):
        return jnp.pad(a, ((0, 0), (0, W - a.shape[1])))

    bias_row = jnp.concatenate(
        [b0, b1, jnp.zeros((W - D0 - D1,), jnp.float32)])[None, :]
    rows = D0 + D1 + B + 1
    packed = jnp.concatenate(
        [pad_w(w0), pad_w(w1), pad_w(relation_features.reshape(B, R * L)),
         bias_row, jnp.zeros(((-rows) % 8, W), jnp.float32)], axis=0)
    packed = pltpu.with_memory_space_constraint(packed, pltpu.MemorySpace.VMEM)

    adj_specs = [
        pl.BlockSpec((1, 1, N, N), (lambda b, rr=r: (b, rr, 0, 0)))
        for r in range(R)
    ]
    return pl.pallas_call(
        _make_body(R, L, B, Din, D0, D1),
        out_shape=jax.ShapeDtypeStruct((B, N, D1), node_features.dtype),
        grid=(B,),
        in_specs=[pl.BlockSpec((1, N, Din), lambda b: (b, 0, 0))] + adj_specs + [
            pl.BlockSpec(packed.shape, lambda b: (0, 0)),
        ],
        out_specs=pl.BlockSpec((1, N, D1), lambda b: (b, 0, 0)),
        scratch_shapes=[
            pltpu.VMEM((Din, R * D0), _CD),
            pltpu.VMEM((D0, R * D1), _CD),
            pltpu.VMEM((B, R * D0), jnp.float32),
            pltpu.VMEM((B, R * D1), jnp.float32),
        ],
        compiler_params=pltpu.CompilerParams(
            dimension_semantics=("arbitrary",),
            vmem_limit_bytes=int((64 << 20) * 0.75)),
    )(node_features, *([adj] * R), packed)
